# baseline (device time: 66834 ns/iter reference)
import jax
import jax.numpy as jnp
from jax import lax
from jax.experimental import pallas as pl
from jax.experimental.pallas import tpu as pltpu

N_DEV = 32
K = 8

_NEG = -3.0e38


def _topk_rows(v, width, k):
    m = v.shape[0]
    col = lax.broadcasted_iota(jnp.int32, (m, width), 1)
    out = []
    for _ in range(k):
        mx = jnp.max(v, axis=1, keepdims=True)
        hit = v == mx
        first = jnp.min(jnp.where(hit, col, width), axis=1, keepdims=True)
        v = jnp.where(col == first, _NEG, v)
        out.append(mx)
    return jnp.concatenate(out, axis=1)


def kernel(x):
    m, n = x.shape

    def body(x_ref, o_ref, cand_ref, recv_ref, send_sems, recv_sems):
        my = lax.axis_index("i")

        barrier = pltpu.get_barrier_semaphore()
        for j in range(1, N_DEV):
            peer = (my + j) % N_DEV
            pl.semaphore_signal(
                barrier, inc=1,
                device_id=(peer,), device_id_type=pl.DeviceIdType.MESH,
            )
        pl.semaphore_wait(barrier, N_DEV - 1)

        cand = _topk_rows(x_ref[:, :], n, K)
        cand_ref[:, :] = cand
        recv_ref[my, :, :] = cand

        rdmas = []
        for j in range(1, N_DEV):
            peer = (my + j) % N_DEV
            rdma = pltpu.make_async_remote_copy(
                src_ref=cand_ref,
                dst_ref=recv_ref.at[my],
                send_sem=send_sems.at[peer],
                recv_sem=recv_sems.at[my],
                device_id=(peer,),
                device_id_type=pl.DeviceIdType.MESH,
            )
            rdma.start()
            rdmas.append(rdma)

        for j in range(1, N_DEV):
            peer = (my + j) % N_DEV
            recv = pltpu.make_async_remote_copy(
                src_ref=cand_ref,
                dst_ref=recv_ref.at[peer],
                send_sem=send_sems.at[peer],
                recv_sem=recv_sems.at[peer],
                device_id=(peer,),
                device_id_type=pl.DeviceIdType.MESH,
            )
            recv.wait_recv()

        parts = [recv_ref[p, :, :] for p in range(N_DEV)]
        both = jnp.concatenate(parts, axis=1)
        o_ref[:, :] = _topk_rows(both, N_DEV * K, K)

        for r in rdmas:
            r.wait_send()

    return pl.pallas_call(
        body,
        out_shape=jax.ShapeDtypeStruct((m, K), jnp.float32),
        in_specs=[pl.BlockSpec(memory_space=pltpu.VMEM)],
        out_specs=pl.BlockSpec(memory_space=pltpu.VMEM),
        scratch_shapes=[
            pltpu.VMEM((m, K), jnp.float32),
            pltpu.VMEM((N_DEV, m, K), jnp.float32),
            pltpu.SemaphoreType.DMA((N_DEV,)),
            pltpu.SemaphoreType.DMA((N_DEV,)),
        ],
        compiler_params=pltpu.CompilerParams(collective_id=0),
    )(x)


# device time: 15732 ns/iter; 4.2483x vs baseline; 4.2483x over previous
import jax
import jax.numpy as jnp
from jax import lax
from jax.experimental import pallas as pl
from jax.experimental.pallas import tpu as pltpu

N_DEV = 32
K = 8

_NEG = -3.0e38


def _topk_rows(v, k):
    out = []
    for _ in range(k):
        mx = jnp.max(v, axis=1, keepdims=True)
        v = jnp.where(v == mx, _NEG, v)
        out.append(mx)
    return jnp.concatenate(out, axis=1)


def _topk_cols(v, k):
    out = []
    for _ in range(k):
        mx = jnp.max(v, axis=0, keepdims=True)
        v = jnp.where(v == mx, _NEG, v)
        out.append(mx)
    return jnp.concatenate(out, axis=0)


def kernel(x):
    m, n = x.shape

    def body(x_ref, o_ref, cand_ref, recv_ref, send_sem, recv_sem):
        my = lax.axis_index("i")

        barrier = pltpu.get_barrier_semaphore()

        def _signal(j, carry):
            peer = (my + j) % N_DEV
            pl.semaphore_signal(
                barrier, inc=1,
                device_id=(peer,), device_id_type=pl.DeviceIdType.MESH,
            )
            return carry

        lax.fori_loop(1, N_DEV, _signal, 0)
        pl.semaphore_wait(barrier, N_DEV - 1)

        cand = jnp.transpose(_topk_rows(x_ref[:, :], K))
        cand_ref[:, :] = cand
        recv_ref[my, :, :] = cand

        def _send(j, carry):
            peer = (my + j) % N_DEV
            rdma = pltpu.make_async_remote_copy(
                src_ref=cand_ref,
                dst_ref=recv_ref.at[my],
                send_sem=send_sem,
                recv_sem=recv_sem,
                device_id=(peer,),
                device_id_type=pl.DeviceIdType.MESH,
            )
            rdma.start()
            return carry

        lax.fori_loop(1, N_DEV, _send, 0)

        drain = pltpu.make_async_remote_copy(
            src_ref=cand_ref,
            dst_ref=recv_ref.at[my],
            send_sem=send_sem,
            recv_sem=recv_sem,
            device_id=(my,),
            device_id_type=pl.DeviceIdType.MESH,
        )

        def _drain(j, carry):
            drain.wait_recv()
            drain.wait_send()
            return carry

        lax.fori_loop(1, N_DEV, _drain, 0)

        allc = recv_ref[:, :, :].reshape(N_DEV * K, m)
        o_ref[:, :] = jnp.transpose(_topk_cols(allc, K))

    return pl.pallas_call(
        body,
        out_shape=jax.ShapeDtypeStruct((m, K), jnp.float32),
        in_specs=[pl.BlockSpec(memory_space=pltpu.VMEM)],
        out_specs=pl.BlockSpec(memory_space=pltpu.VMEM),
        scratch_shapes=[
            pltpu.VMEM((K, m), jnp.float32),
            pltpu.VMEM((N_DEV, K, m), jnp.float32),
            pltpu.SemaphoreType.DMA,
            pltpu.SemaphoreType.DMA,
        ],
        compiler_params=pltpu.CompilerParams(collective_id=0),
    )(x)


# device time: 14616 ns/iter; 4.5727x vs baseline; 1.0764x over previous
import jax
import jax.numpy as jnp
from jax import lax
from jax.experimental import pallas as pl
from jax.experimental.pallas import tpu as pltpu

N_DEV = 32
K = 8

_NEG = -3.0e38


def _topk_rows(v, k):
    out = []
    for _ in range(k):
        mx = jnp.max(v, axis=1, keepdims=True)
        v = jnp.where(v == mx, _NEG, v)
        out.append(mx)
    return jnp.concatenate(out, axis=1)


def _topk_cols(v, k):
    out = []
    for _ in range(k):
        mx = jnp.max(v, axis=0, keepdims=True)
        v = jnp.where(v == mx, _NEG, v)
        out.append(mx)
    return jnp.concatenate(out, axis=0)


def kernel(x):
    m, n = x.shape

    def body(x_ref, o_ref, cand_ref, recv_ref, send_sem, recv_sem):
        my = lax.axis_index("i")

        barrier = pltpu.get_barrier_semaphore()

        def _signal(j, carry):
            peer = (my + j) % N_DEV
            pl.semaphore_signal(
                barrier, inc=1,
                device_id=(peer,), device_id_type=pl.DeviceIdType.MESH,
            )
            return carry

        lax.fori_loop(1, N_DEV, _signal, 0)

        cand = jnp.transpose(_topk_rows(x_ref[:, :], K))
        cand_ref[:, :] = cand
        recv_ref[my, :, :] = cand

        pl.semaphore_wait(barrier, N_DEV - 1)

        def _send(j, carry):
            peer = (my + j) % N_DEV
            rdma = pltpu.make_async_remote_copy(
                src_ref=cand_ref,
                dst_ref=recv_ref.at[my],
                send_sem=send_sem,
                recv_sem=recv_sem,
                device_id=(peer,),
                device_id_type=pl.DeviceIdType.MESH,
            )
            rdma.start()
            return carry

        lax.fori_loop(1, N_DEV, _send, 0)

        drain = pltpu.make_async_remote_copy(
            src_ref=cand_ref,
            dst_ref=recv_ref.at[my],
            send_sem=send_sem,
            recv_sem=recv_sem,
            device_id=(my,),
            device_id_type=pl.DeviceIdType.MESH,
        )

        def _drain(j, carry):
            drain.wait_recv()
            drain.wait_send()
            return carry

        lax.fori_loop(1, N_DEV, _drain, 0)

        allc = recv_ref[:, :, :].reshape(N_DEV * K, m)
        o_ref[:, :] = jnp.transpose(_topk_cols(allc, K))

    return pl.pallas_call(
        body,
        out_shape=jax.ShapeDtypeStruct((m, K), jnp.float32),
        in_specs=[pl.BlockSpec(memory_space=pltpu.VMEM)],
        out_specs=pl.BlockSpec(memory_space=pltpu.VMEM),
        scratch_shapes=[
            pltpu.VMEM((K, m), jnp.float32),
            pltpu.VMEM((N_DEV, K, m), jnp.float32),
            pltpu.SemaphoreType.DMA,
            pltpu.SemaphoreType.DMA,
        ],
        compiler_params=pltpu.CompilerParams(collective_id=0),
    )(x)
